# Initial kernel scaffold; baseline (speedup 1.0000x reference)
import jax
import jax.numpy as jnp
from jax.experimental import pallas as pl


def kernel(x, weight, e_score_correction_bias):
    raise NotImplementedError("write your pallas kernel here")


# speedup vs baseline: 4.3010x; 4.3010x over previous
import jax
import jax.numpy as jnp
from jax.experimental import pallas as pl
from jax.experimental.pallas import tpu as pltpu

NE = 64
NG = 8
GSZ = NE // NG
TG = 4
TK = 8
SCALE = 2.5
HID = 4096
TOKENS = 16384
BT = 256


def _router_block(x_ref, w_ref, b_ref, idx_ref, wgt_ref):
    logits = jax.lax.dot_general(
        w_ref[...], x_ref[...],
        dimension_numbers=(((1,), (1,)), ((), ())),
        preferred_element_type=jnp.float32,
    )
    s = 1.0 / (1.0 + jnp.exp(-logits))
    sb = s + b_ref[...]

    g3 = sb.reshape(NG, GSZ, BT)
    m1 = jnp.max(g3, axis=1)
    eqm = g3 == m1[:, None, :]
    cnt = jnp.sum(eqm.astype(jnp.float32), axis=1)
    m2 = jnp.max(jnp.where(eqm, -jnp.inf, g3), axis=1)
    m2 = jnp.where(cnt > 1.0, m1, m2)
    gs = m1 + m2

    rank = jnp.zeros((NG, BT), jnp.int32)
    row = jax.lax.broadcasted_iota(jnp.int32, (NG, BT), 0)
    for d in range(1, NG):
        rolled = jnp.concatenate([gs[d:], gs[:d]], axis=0)
        gt = (rolled > gs).astype(jnp.int32)
        tie = jnp.where((rolled == gs) & (row >= NG - d), 1, 0)
        rank = rank + gt + tie
    gmask = rank < TG

    mask64 = jnp.broadcast_to(gmask[:, None, :], (NG, GSZ, BT)).reshape(NE, BT)
    ms = jnp.where(mask64, sb, 0.0)

    riota = jax.lax.broadcasted_iota(jnp.int32, (NE, BT), 0)
    idxs = []
    wts = []
    for _ in range(TK):
        m = jnp.max(ms, axis=0, keepdims=True)
        cand = jnp.where(ms == m, riota, NE)
        ik = jnp.min(cand, axis=0, keepdims=True)
        sel = riota == ik
        wk = jnp.max(jnp.where(sel, s, -jnp.inf), axis=0, keepdims=True)
        ms = jnp.where(sel, -jnp.inf, ms)
        idxs.append(ik)
        wts.append(wk)
    idx = jnp.concatenate(idxs, axis=0)
    w = jnp.concatenate(wts, axis=0)

    denom = jnp.sum(w, axis=0, keepdims=True) + 1e-20
    w = w / denom * SCALE

    idx_ref[...] = idx
    wgt_ref[...] = w


def kernel(x, weight, e_score_correction_bias):
    b = e_score_correction_bias.reshape(NE, 1).astype(jnp.float32)
    grid = (TOKENS // BT,)
    idx_t, w_t = pl.pallas_call(
        _router_block,
        grid=grid,
        in_specs=[
            pl.BlockSpec((BT, HID), lambda i: (i, 0)),
            pl.BlockSpec((NE, HID), lambda i: (0, 0)),
            pl.BlockSpec((NE, 1), lambda i: (0, 0)),
        ],
        out_specs=[
            pl.BlockSpec((TK, BT), lambda i: (0, i)),
            pl.BlockSpec((TK, BT), lambda i: (0, i)),
        ],
        out_shape=[
            jax.ShapeDtypeStruct((TK, TOKENS), jnp.int32),
            jax.ShapeDtypeStruct((TK, TOKENS), jnp.float32),
        ],
    )(x.astype(jnp.float32), weight.astype(jnp.float32), b)
    return idx_t.T, w_t.T


# speedup vs baseline: 5.1846x; 1.2054x over previous
import jax
import jax.numpy as jnp
from jax.experimental import pallas as pl
from jax.experimental.pallas import tpu as pltpu

NE = 64
NG = 8
GSZ = NE // NG
TG = 4
TK = 8
SCALE = 2.5
HID = 4096
TOKENS = 16384
BT = 512


def _router_block(x_ref, w_ref, b_ref, idx_ref, wgt_ref):
    logits = jax.lax.dot_general(
        w_ref[...], x_ref[...],
        dimension_numbers=(((1,), (1,)), ((), ())),
        preferred_element_type=jnp.float32,
    )
    s = 1.0 / (1.0 + jnp.exp(-logits))
    sb = s + b_ref[...]

    g3 = sb.reshape(NG, GSZ, BT)
    m1 = jnp.max(g3, axis=1)
    eqm = g3 == m1[:, None, :]
    cnt = jnp.sum(eqm.astype(jnp.float32), axis=1)
    m2 = jnp.max(jnp.where(eqm, -jnp.inf, g3), axis=1)
    m2 = jnp.where(cnt > 1.0, m1, m2)
    gs = m1 + m2

    rank = jnp.zeros((NG, BT), jnp.int32)
    row = jax.lax.broadcasted_iota(jnp.int32, (NG, BT), 0)
    for d in range(1, NG):
        rolled = jnp.concatenate([gs[d:], gs[:d]], axis=0)
        gt = (rolled > gs).astype(jnp.int32)
        tie = jnp.where((rolled == gs) & (row >= NG - d), 1, 0)
        rank = rank + gt + tie
    gmask = rank < TG

    mask64 = jnp.broadcast_to(gmask[:, None, :], (NG, GSZ, BT)).reshape(NE, BT)
    ms = jnp.where(mask64, sb, 0.0)

    riota = jax.lax.broadcasted_iota(jnp.int32, (NE, BT), 0)
    idxs = []
    wts = []
    for _ in range(TK):
        m = jnp.max(ms, axis=0, keepdims=True)
        cand = jnp.where(ms == m, riota, NE)
        ik = jnp.min(cand, axis=0, keepdims=True)
        sel = riota == ik
        wk = jnp.max(jnp.where(sel, s, -jnp.inf), axis=0, keepdims=True)
        ms = jnp.where(sel, -jnp.inf, ms)
        idxs.append(ik)
        wts.append(wk)
    idx = jnp.concatenate(idxs, axis=0)
    w = jnp.concatenate(wts, axis=0)

    denom = jnp.sum(w, axis=0, keepdims=True) + 1e-20
    w = w / denom * SCALE

    idx_ref[...] = idx
    wgt_ref[...] = w


def kernel(x, weight, e_score_correction_bias):
    b = e_score_correction_bias.reshape(NE, 1).astype(jnp.float32)
    grid = (TOKENS // BT,)
    idx_t, w_t = pl.pallas_call(
        _router_block,
        grid=grid,
        in_specs=[
            pl.BlockSpec((BT, HID), lambda i: (i, 0)),
            pl.BlockSpec((NE, HID), lambda i: (0, 0)),
            pl.BlockSpec((NE, 1), lambda i: (0, 0)),
        ],
        out_specs=[
            pl.BlockSpec((TK, BT), lambda i: (0, i)),
            pl.BlockSpec((TK, BT), lambda i: (0, i)),
        ],
        out_shape=[
            jax.ShapeDtypeStruct((TK, TOKENS), jnp.int32),
            jax.ShapeDtypeStruct((TK, TOKENS), jnp.float32),
        ],
    )(x.astype(jnp.float32), weight.astype(jnp.float32), b)
    return idx_t.T, w_t.T


# speedup vs baseline: 5.7320x; 1.1056x over previous
import jax
import jax.numpy as jnp
from jax.experimental import pallas as pl
from jax.experimental.pallas import tpu as pltpu

NE = 64
NG = 8
GSZ = NE // NG
TG = 4
TK = 8
SCALE = 2.5
HID = 4096
TOKENS = 16384
BT = 1024


def _router_block(x_ref, w_ref, b_ref, idx_ref, wgt_ref):
    logits = jax.lax.dot_general(
        w_ref[...], x_ref[...],
        dimension_numbers=(((1,), (1,)), ((), ())),
        preferred_element_type=jnp.float32,
    )
    s = 1.0 / (1.0 + jnp.exp(-logits))
    sb = s + b_ref[...]

    g3 = sb.reshape(NG, GSZ, BT)
    m1 = jnp.max(g3, axis=1)
    eqm = g3 == m1[:, None, :]
    cnt = jnp.sum(eqm.astype(jnp.float32), axis=1)
    m2 = jnp.max(jnp.where(eqm, -jnp.inf, g3), axis=1)
    m2 = jnp.where(cnt > 1.0, m1, m2)
    gs = m1 + m2

    rank = jnp.zeros((NG, BT), jnp.int32)
    row = jax.lax.broadcasted_iota(jnp.int32, (NG, BT), 0)
    for d in range(1, NG):
        rolled = jnp.concatenate([gs[d:], gs[:d]], axis=0)
        gt = (rolled > gs).astype(jnp.int32)
        tie = jnp.where((rolled == gs) & (row >= NG - d), 1, 0)
        rank = rank + gt + tie
    gmask = rank < TG

    mask64 = jnp.broadcast_to(gmask[:, None, :], (NG, GSZ, BT)).reshape(NE, BT)
    ms = jnp.where(mask64, sb, 0.0)

    riota = jax.lax.broadcasted_iota(jnp.int32, (NE, BT), 0)
    idxs = []
    wts = []
    for _ in range(TK):
        m = jnp.max(ms, axis=0, keepdims=True)
        cand = jnp.where(ms == m, riota, NE)
        ik = jnp.min(cand, axis=0, keepdims=True)
        sel = riota == ik
        wk = jnp.max(jnp.where(sel, s, -jnp.inf), axis=0, keepdims=True)
        ms = jnp.where(sel, -jnp.inf, ms)
        idxs.append(ik)
        wts.append(wk)
    idx = jnp.concatenate(idxs, axis=0)
    w = jnp.concatenate(wts, axis=0)

    denom = jnp.sum(w, axis=0, keepdims=True) + 1e-20
    w = w / denom * SCALE

    idx_ref[...] = idx
    wgt_ref[...] = w


def kernel(x, weight, e_score_correction_bias):
    b = e_score_correction_bias.reshape(NE, 1).astype(jnp.float32)
    grid = (TOKENS // BT,)
    idx_t, w_t = pl.pallas_call(
        _router_block,
        grid=grid,
        in_specs=[
            pl.BlockSpec((BT, HID), lambda i: (i, 0)),
            pl.BlockSpec((NE, HID), lambda i: (0, 0)),
            pl.BlockSpec((NE, 1), lambda i: (0, 0)),
        ],
        out_specs=[
            pl.BlockSpec((TK, BT), lambda i: (0, i)),
            pl.BlockSpec((TK, BT), lambda i: (0, i)),
        ],
        out_shape=[
            jax.ShapeDtypeStruct((TK, TOKENS), jnp.int32),
            jax.ShapeDtypeStruct((TK, TOKENS), jnp.float32),
        ],
    )(x.astype(jnp.float32), weight.astype(jnp.float32), b)
    return idx_t.T, w_t.T
